# baseline (speedup 1.0000x reference)
import jax
import jax.numpy as jnp
from jax import lax
from jax.experimental import pallas as pl
from jax.experimental.pallas import tpu as pltpu
from jax.experimental.pallas import tpu_sc as plsc

_BATCH = 16384
_HIST = 50
_DIM = 64
_B_TOT = _BATCH * _HIST
_NC = 2
_NS = 16
_NW = _NC * _NS
_B_PER_W = _B_TOT // _NW
_CHUNK = 1024
_N_CHUNK = _B_PER_W // _CHUNK


def _emb_body(table_hbm, idx_hbm, out_hbm, idx_v, rows_v, sem):
    wid = lax.axis_index("s") * _NC + lax.axis_index("c")
    base = wid * _B_PER_W

    def step(i, carry):
        off = base + i * _CHUNK
        pltpu.sync_copy(idx_hbm.at[pl.ds(off, _CHUNK)], idx_v)
        pltpu.async_copy(table_hbm.at[idx_v], rows_v, sem).wait()
        pltpu.sync_copy(rows_v, out_hbm.at[pl.ds(off, _CHUNK)])
        return carry

    lax.fori_loop(0, _N_CHUNK, step, 0)


def kernel(token_ids, param):
    idx = token_ids.reshape(_B_TOT).astype(jnp.int32)
    mesh = plsc.VectorSubcoreMesh(core_axis_name="c", subcore_axis_name="s")
    out = pl.kernel(
        _emb_body,
        out_type=jax.ShapeDtypeStruct((_B_TOT, _DIM), jnp.float32),
        mesh=mesh,
        compiler_params=pltpu.CompilerParams(use_tc_tiling_on_sc=False),
        scratch_types=[
            pltpu.VMEM((_CHUNK,), jnp.int32),
            pltpu.VMEM((_CHUNK, _DIM), jnp.float32),
            pltpu.SemaphoreType.DMA,
        ],
    )(param, idx)
    return out.reshape(_BATCH, _HIST, _DIM)


# speedup vs baseline: 1.0187x; 1.0187x over previous
import jax
import jax.numpy as jnp
from jax import lax
from jax.experimental import pallas as pl
from jax.experimental.pallas import tpu as pltpu
from jax.experimental.pallas import tpu_sc as plsc

_BATCH = 16384
_HIST = 50
_DIM = 64
_B_TOT = _BATCH * _HIST
_NC = 2
_NS = 16
_NW = _NC * _NS
_B_PER_W = _B_TOT // _NW
_CHUNK = 400
_N_CHUNK = _B_PER_W // _CHUNK
_NBUF = 4


def _emb_body(table_hbm, idx_hbm, out_hbm, idx_v, r0, r1, r2, r3,
              g0, g1, g2, g3, s0, s1, s2, s3):
    rows = (r0, r1, r2, r3)
    gsem = (g0, g1, g2, g3)
    ssem = (s0, s1, s2, s3)
    wid = lax.axis_index("s") * _NC + lax.axis_index("c")
    base = wid * _B_PER_W

    pltpu.sync_copy(idx_hbm.at[pl.ds(base, _B_PER_W)], idx_v)

    def start_gather(i, b):
        return pltpu.async_copy(
            table_hbm.at[idx_v.at[pl.ds(i * _CHUNK, _CHUNK)]], rows[b], gsem[b])

    for b in range(_NBUF):
        start_gather(b, b)

    def step(i, b, last):
        pltpu.make_async_copy(
            table_hbm.at[idx_v.at[pl.ds(0, _CHUNK)]], rows[b], gsem[b]).wait()
        st = pltpu.async_copy(
            rows[b], out_hbm.at[pl.ds(base + i * _CHUNK, _CHUNK)], ssem[b])
        if not last:
            st.wait()
            start_gather(i + _NBUF, b)
        return st

    def outer(j, carry):
        for b in range(_NBUF):
            step(j * _NBUF + b, b, last=False)
        return carry

    lax.fori_loop(0, _N_CHUNK // _NBUF - 1, outer, 0)
    tail = []
    for b in range(_NBUF):
        tail.append(step(_N_CHUNK - _NBUF + b, b, last=True))
    for st in tail:
        st.wait()


def kernel(token_ids, param):
    idx = token_ids.reshape(_B_TOT).astype(jnp.int32)
    mesh = plsc.VectorSubcoreMesh(core_axis_name="c", subcore_axis_name="s")
    out = pl.kernel(
        _emb_body,
        out_type=jax.ShapeDtypeStruct((_B_TOT, _DIM), jnp.float32),
        mesh=mesh,
        compiler_params=pltpu.CompilerParams(use_tc_tiling_on_sc=False),
        scratch_types=[
            pltpu.VMEM((_B_PER_W,), jnp.int32),
            pltpu.VMEM((_CHUNK, _DIM), jnp.float32),
            pltpu.VMEM((_CHUNK, _DIM), jnp.float32),
            pltpu.VMEM((_CHUNK, _DIM), jnp.float32),
            pltpu.VMEM((_CHUNK, _DIM), jnp.float32),
            pltpu.SemaphoreType.DMA,
            pltpu.SemaphoreType.DMA,
            pltpu.SemaphoreType.DMA,
            pltpu.SemaphoreType.DMA,
            pltpu.SemaphoreType.DMA,
            pltpu.SemaphoreType.DMA,
            pltpu.SemaphoreType.DMA,
            pltpu.SemaphoreType.DMA,
        ],
    )(param, idx)
    return out.reshape(_BATCH, _HIST, _DIM)
